# baseline (speedup 1.0000x reference)
import functools

import jax
import jax.numpy as jnp
from jax import lax
from jax.experimental import pallas as pl
from jax.experimental.pallas import tpu as pltpu
from jax.experimental.pallas import tpu_sc as plsc

N_VOCAB = 1000000
K_DIM = 32
BATCH = 4096
N_FIELDS = 26

_NC = 2
_NS = 16
_NW = _NC * _NS
_ROWS_PER_W = BATCH // _NW
_IDX_PER_W = _ROWS_PER_W * N_FIELDS
_STRIPS = _ROWS_PER_W // 16

_mesh = plsc.VectorSubcoreMesh(core_axis_name="c", subcore_axis_name="s")


@functools.partial(
    pl.kernel,
    out_type=jax.ShapeDtypeStruct((BATCH,), jnp.float32),
    mesh=_mesh,
    compiler_params=pltpu.CompilerParams(
        needs_layout_passes=False, use_tc_tiling_on_sc=False),
    scratch_types=[
        pltpu.VMEM((_IDX_PER_W,), jnp.int32),
        pltpu.VMEM((_IDX_PER_W,), jnp.float32),
        pltpu.VMEM((_IDX_PER_W,), jnp.float32),
        pltpu.VMEM((_IDX_PER_W,), jnp.float32),
        pltpu.VMEM((_ROWS_PER_W,), jnp.float32),
        pltpu.VMEM((16,), jnp.float32),
        pltpu.SemaphoreType.DMA,
        pltpu.SemaphoreType.DMA,
        pltpu.SemaphoreType.DMA,
    ],
)
def _fm_sc(idx_hbm, w1_hbm, v_hbm, w0_hbm, out_hbm,
           idx_v, buf_a, buf_b, w1_v, acc_v, w0_v, sem_a, sem_b, sem_w):
    wid = lax.axis_index("s") * _NC + lax.axis_index("c")
    base = wid * _IDX_PER_W

    pltpu.sync_copy(w0_hbm, w0_v)
    pltpu.sync_copy(idx_hbm.at[pl.ds(base, _IDX_PER_W)], idx_v)
    cp_w = pltpu.async_copy(w1_hbm.at[idx_v], w1_v, sem_w)

    bufs = (buf_a, buf_b)
    sems = (sem_a, sem_b)

    def stage(k):
        return pltpu.async_copy(
            v_hbm.at[pl.ds(k * N_VOCAB, N_VOCAB)].at[idx_v],
            bufs[k % 2], sems[k % 2])

    cp = {0: stage(0), 1: stage(1)}

    lane = lax.broadcasted_iota(jnp.int32, (16,), 0)
    lane_f = lane * N_FIELDS
    zero16 = jnp.zeros((16,), jnp.float32)

    cp_w.wait()
    w0 = w0_v[...]

    def lv_body(c, _):
        def f_body(f, l):
            return l + plsc.load_gather(w1_v, [c * (16 * N_FIELDS) + lane_f + f])
        lv = lax.fori_loop(0, N_FIELDS, f_body, zero16)
        acc_v[pl.ds(c * 16, 16)] = (lv + w0) * 2.0
        return 0

    lax.fori_loop(0, _STRIPS, lv_body, 0)

    for k in range(K_DIM):
        cp[k].wait()
        buf = bufs[k % 2]

        def strip_body(c, _):
            o = c * (16 * N_FIELDS)

            def f_body(f, carry):
                s, q = carry
                val = plsc.load_gather(buf, [o + lane_f + f])
                return (s + val, q + val * val)

            s, q = lax.fori_loop(0, N_FIELDS, f_body, (zero16, zero16))
            acc_v[pl.ds(c * 16, 16)] = acc_v[pl.ds(c * 16, 16)] + s * s - q
            return 0

        lax.fori_loop(0, _STRIPS, strip_body, 0)
        if k + 2 < K_DIM:
            cp[k + 2] = stage(k + 2)

    def out_body(c, _):
        acc_v[pl.ds(c * 16, 16)] = acc_v[pl.ds(c * 16, 16)] * 0.5
        return 0

    lax.fori_loop(0, _STRIPS, out_body, 0)
    pltpu.sync_copy(acc_v, out_hbm.at[pl.ds(wid * _ROWS_PER_W, _ROWS_PER_W)])


def kernel(inputs, W1, V, W0):
    idx = inputs.reshape(-1).astype(jnp.int32)
    w0b = jnp.broadcast_to(W0, (16,))
    vflat = jnp.swapaxes(V, 0, 1).reshape(-1)
    w1f = jnp.swapaxes(W1, 0, 1).reshape(-1)
    return _fm_sc(idx, w1f, vflat, w0b).reshape(BATCH, 1)


# speedup vs baseline: 9.9295x; 9.9295x over previous
import functools

import jax
import jax.numpy as jnp
from jax import lax
from jax.experimental import pallas as pl
from jax.experimental.pallas import tpu as pltpu
from jax.experimental.pallas import tpu_sc as plsc

N_VOCAB = 1000000
K_DIM = 32
BATCH = 4096
N_FIELDS = 26

_SLABS = 4
_SLAB = 1 << 18
_NC = 2
_NS = 16
_NW = _NC * _NS
_ROWS_PER_W = BATCH // _NW
_IDX_PER_W = _ROWS_PER_W * N_FIELDS
_STRIPS = _ROWS_PER_W // 16
_IDX_PER_STRIP = 16 * N_FIELDS

_mesh = plsc.VectorSubcoreMesh(core_axis_name="c", subcore_axis_name="s")

_TW = 2048
_TGRID = _SLAB // _TW
_W1B = 8192


def _prep_body(v0_ref, v1_ref, v2_ref, v3_ref, vp_ref):
    stacked = jnp.concatenate(
        [ref[...] for ref in (v0_ref, v1_ref, v2_ref, v3_ref)], axis=0)
    vp_ref[...] = stacked.T


_tc_prep = pl.pallas_call(
    _prep_body,
    grid=(_TGRID,),
    in_specs=[
        pl.BlockSpec(
            (K_DIM, _TW),
            lambda g, c=c: (0, jnp.minimum(c * _TGRID + g,
                                           N_VOCAB // _TW)))
        for c in range(_SLABS)
    ],
    out_specs=pl.BlockSpec((_TW, _SLABS * K_DIM), lambda g: (g, 0)),
    out_shape=jax.ShapeDtypeStruct((_SLAB, _SLABS * K_DIM), jnp.float32),
)


def _w1_body(w1t_ref, w1_ref):
    w1_ref[...] = w1t_ref[0, :]


_w1_prep = pl.pallas_call(
    _w1_body,
    grid=((N_VOCAB + _W1B - 1) // _W1B,),
    in_specs=[pl.BlockSpec((1, _W1B), lambda g: (0, g))],
    out_specs=pl.BlockSpec((_W1B,), lambda g: (g,)),
    out_shape=jax.ShapeDtypeStruct((N_VOCAB,), jnp.float32),
)


@functools.partial(
    pl.kernel,
    out_type=jax.ShapeDtypeStruct((BATCH,), jnp.float32),
    mesh=_mesh,
    compiler_params=pltpu.CompilerParams(
        needs_layout_passes=False, use_tc_tiling_on_sc=False),
    scratch_types=[
        pltpu.VMEM((_IDX_PER_W,), jnp.int32),
        pltpu.VMEM((_IDX_PER_STRIP, 4 * K_DIM), jnp.float32),
        pltpu.VMEM((_IDX_PER_STRIP, 4 * K_DIM), jnp.float32),
        pltpu.VMEM((_IDX_PER_STRIP,), jnp.int32),
        pltpu.VMEM((_IDX_PER_STRIP,), jnp.int32),
        pltpu.VMEM((_IDX_PER_W,), jnp.float32),
        pltpu.VMEM((_ROWS_PER_W,), jnp.float32),
        pltpu.VMEM((16,), jnp.float32),
        pltpu.SemaphoreType.DMA,
        pltpu.SemaphoreType.DMA,
        pltpu.SemaphoreType.DMA,
    ],
)
def _fm_sc(idx_hbm, w1_hbm, vp_hbm, w0_hbm, out_hbm,
           idx_v, buf_a, buf_b, qid_a, qid_b, w1_v, out_v, w0_v,
           sem_a, sem_b, sem_w):
    wid = lax.axis_index("s") * _NC + lax.axis_index("c")
    base = wid * _IDX_PER_W

    pltpu.sync_copy(w0_hbm, w0_v)
    pltpu.sync_copy(idx_hbm.at[pl.ds(base, _IDX_PER_W)], idx_v)
    cp_w = pltpu.async_copy(w1_hbm.at[idx_v], w1_v, sem_w)

    bufs = (buf_a, buf_b)
    qids = (qid_a, qid_b)
    sems = (sem_a, sem_b)

    def stage(c):
        qid = qids[c % 2]

        def qbody(i, _):
            o = i * 16
            qid[pl.ds(o, 16)] = jnp.bitwise_and(
                idx_v[pl.ds(c * _IDX_PER_STRIP + o, 16)], _SLAB - 1)
            return 0

        lax.fori_loop(0, _IDX_PER_STRIP // 16, qbody, 0)
        return pltpu.async_copy(vp_hbm.at[qid], bufs[c % 2], sems[c % 2])

    cp = {0: stage(0)}
    cp_w.wait()

    w0 = w0_v[...]
    lane = lax.broadcasted_iota(jnp.int32, (16,), 0)
    lane26 = lane * N_FIELDS
    zero16 = jnp.zeros((16,), jnp.float32)

    for c in range(_STRIPS):
        cp[c].wait()
        if c + 1 < _STRIPS:
            cp[c + 1] = stage(c + 1)
        buf = bufs[c % 2]

        acc = zero16
        lv = zero16
        for h in range(2):
            def f_body(f, carry):
                s = list(carry[0])
                q = carry[1]
                l = carry[2]
                idx0 = lane26 + f
                raw = plsc.load_gather(idx_v, [c * _IDX_PER_STRIP + idx0])
                off = lax.shift_left(
                    lax.shift_right_logical(raw, 18), 5) + h * 16
                for k in range(16):
                    val = plsc.load_gather(buf, [idx0, off + k])
                    q = q + val * val
                    s[k] = s[k] + val
                if h == 0:
                    l = l + plsc.load_gather(
                        w1_v, [c * _IDX_PER_STRIP + idx0])
                return (tuple(s), q, l)

            s, q, lv = lax.fori_loop(
                0, N_FIELDS, f_body, ((zero16,) * 16, zero16, lv))
            acc = acc - q
            for k in range(16):
                acc = acc + s[k] * s[k]

        out_v[pl.ds(c * 16, 16)] = lv + w0 + 0.5 * acc

    pltpu.sync_copy(out_v, out_hbm.at[pl.ds(wid * _ROWS_PER_W, _ROWS_PER_W)])


def kernel(inputs, W1, V, W0):
    idx = inputs.reshape(-1).astype(jnp.int32)
    w0b = jnp.broadcast_to(W0, (16,))
    vt = jnp.swapaxes(V, 0, 1)
    vp = _tc_prep(vt, vt, vt, vt)
    w1f = _w1_prep(jnp.swapaxes(W1, 0, 1))
    out = _fm_sc(idx, w1f, vp, w0b)
    return out.reshape(BATCH, 1)


# speedup vs baseline: 12.4621x; 1.2551x over previous
import functools

import jax
import jax.numpy as jnp
from jax import lax
from jax.experimental import pallas as pl
from jax.experimental.pallas import tpu as pltpu
from jax.experimental.pallas import tpu_sc as plsc

N_VOCAB = 1000000
K_DIM = 32
BATCH = 4096
N_FIELDS = 26

_SLABS = 4
_SLAB = 1 << 18
_NC = 2
_NS = 16
_NW = _NC * _NS
_ROWS_PER_W = BATCH // _NW
_IDX_PER_W = _ROWS_PER_W * N_FIELDS
_STRIPS = _ROWS_PER_W // 16
_IDX_PER_STRIP = 16 * N_FIELDS

_mesh = plsc.VectorSubcoreMesh(core_axis_name="c", subcore_axis_name="s")

_TW = 4096
_TGRID = _SLAB // _TW


def _prep_body(v0_ref, v1_ref, v2_ref, v3_ref, vp_ref):
    stacked = jnp.concatenate(
        [ref[...] for ref in (v0_ref, v1_ref, v2_ref, v3_ref)], axis=0)
    vp_ref[...] = stacked.T


_tc_prep = pl.pallas_call(
    _prep_body,
    grid=(_TGRID,),
    in_specs=[
        pl.BlockSpec(
            (K_DIM, _TW),
            lambda g, c=c: (0, jnp.minimum(c * _TGRID + g,
                                           N_VOCAB // _TW)))
        for c in range(_SLABS)
    ],
    out_specs=pl.BlockSpec((_TW, _SLABS * K_DIM), lambda g: (g, 0)),
    out_shape=jax.ShapeDtypeStruct((_SLAB, _SLABS * K_DIM), jnp.float32),
)


@functools.partial(
    pl.kernel,
    out_type=jax.ShapeDtypeStruct((BATCH,), jnp.float32),
    mesh=_mesh,
    compiler_params=pltpu.CompilerParams(
        needs_layout_passes=False, use_tc_tiling_on_sc=False),
    scratch_types=[
        pltpu.VMEM((_IDX_PER_W,), jnp.int32),
        pltpu.VMEM((_IDX_PER_STRIP, 4 * K_DIM), jnp.float32),
        pltpu.VMEM((_IDX_PER_STRIP, 4 * K_DIM), jnp.float32),
        pltpu.VMEM((_IDX_PER_STRIP,), jnp.int32),
        pltpu.VMEM((_IDX_PER_STRIP,), jnp.int32),
        pltpu.VMEM((_IDX_PER_W,), jnp.float32),
        pltpu.VMEM((_ROWS_PER_W,), jnp.float32),
        pltpu.VMEM((16,), jnp.float32),
        pltpu.SemaphoreType.DMA,
        pltpu.SemaphoreType.DMA,
        pltpu.SemaphoreType.DMA,
    ],
)
def _fm_sc(idx_hbm, w1_hbm, vp_hbm, w0_hbm, out_hbm,
           idx_v, buf_a, buf_b, qid_a, qid_b, w1_v, out_v, w0_v,
           sem_a, sem_b, sem_w):
    wid = lax.axis_index("s") * _NC + lax.axis_index("c")
    base = wid * _IDX_PER_W

    pltpu.sync_copy(w0_hbm, w0_v)
    pltpu.sync_copy(idx_hbm.at[pl.ds(base, _IDX_PER_W)], idx_v)
    cp_w = pltpu.async_copy(w1_hbm.at[idx_v], w1_v, sem_w)

    bufs = (buf_a, buf_b)
    qids = (qid_a, qid_b)
    sems = (sem_a, sem_b)

    def stage(c):
        qid = qids[c % 2]

        def qbody(i, _):
            o = i * 16
            qid[pl.ds(o, 16)] = jnp.bitwise_and(
                idx_v[pl.ds(c * _IDX_PER_STRIP + o, 16)], _SLAB - 1)
            return 0

        lax.fori_loop(0, _IDX_PER_STRIP // 16, qbody, 0)
        return pltpu.async_copy(vp_hbm.at[qid], bufs[c % 2], sems[c % 2])

    cp = {0: stage(0)}
    cp_w.wait()

    w0 = w0_v[...]
    lane = lax.broadcasted_iota(jnp.int32, (16,), 0)
    lane26 = lane * N_FIELDS
    zero16 = jnp.zeros((16,), jnp.float32)

    for c in range(_STRIPS):
        cp[c].wait()
        if c + 1 < _STRIPS:
            cp[c + 1] = stage(c + 1)
        buf = bufs[c % 2]

        acc = zero16
        lv = zero16
        for h in range(2):
            def f_body(f, carry):
                s = list(carry[0])
                q = carry[1]
                l = carry[2]
                idx0 = lane26 + f
                raw = plsc.load_gather(idx_v, [c * _IDX_PER_STRIP + idx0])
                off = lax.shift_left(
                    lax.shift_right_logical(raw, 18), 5) + h * 16
                for k in range(16):
                    val = plsc.load_gather(buf, [idx0, off + k])
                    q = q + val * val
                    s[k] = s[k] + val
                if h == 0:
                    l = l + plsc.load_gather(
                        w1_v, [c * _IDX_PER_STRIP + idx0])
                return (tuple(s), q, l)

            s, q, lv = lax.fori_loop(
                0, N_FIELDS, f_body, ((zero16,) * 16, zero16, lv))
            acc = acc - q
            for k in range(16):
                acc = acc + s[k] * s[k]

        out_v[pl.ds(c * 16, 16)] = lv + w0 + 0.5 * acc

    pltpu.sync_copy(out_v, out_hbm.at[pl.ds(wid * _ROWS_PER_W, _ROWS_PER_W)])


def kernel(inputs, W1, V, W0):
    idx = inputs.reshape(-1).astype(jnp.int32)
    w0b = jnp.broadcast_to(W0, (16,))
    vt = jnp.swapaxes(V, 0, 1)
    vp = _tc_prep(vt, vt, vt, vt)
    w1f = jnp.swapaxes(W1, 0, 1).reshape(-1)
    out = _fm_sc(idx, w1f, vp, w0b)
    return out.reshape(BATCH, 1)


# speedup vs baseline: 12.5034x; 1.0033x over previous
import functools

import jax
import jax.numpy as jnp
from jax import lax
from jax.experimental import pallas as pl
from jax.experimental.pallas import tpu as pltpu
from jax.experimental.pallas import tpu_sc as plsc

N_VOCAB = 1000000
K_DIM = 32
BATCH = 4096
N_FIELDS = 26

_SLABS = 4
_SLAB = 1 << 18
_NC = 2
_NS = 16
_NW = _NC * _NS
_ROWS_PER_W = BATCH // _NW
_IDX_PER_W = _ROWS_PER_W * N_FIELDS
_STRIPS = _ROWS_PER_W // 16
_IDX_PER_STRIP = 16 * N_FIELDS

_mesh = plsc.VectorSubcoreMesh(core_axis_name="c", subcore_axis_name="s")

_TW = 4096
_TGRID = _SLAB // _TW


def _prep_body(v0_ref, v1_ref, v2_ref, v3_ref, vp_ref):
    stacked = jnp.concatenate(
        [ref[...] for ref in (v0_ref, v1_ref, v2_ref, v3_ref)], axis=0)
    vp_ref[...] = stacked.T


_tc_prep = pl.pallas_call(
    _prep_body,
    grid=(_TGRID,),
    in_specs=[
        pl.BlockSpec(
            (K_DIM, _TW),
            lambda g, c=c: (0, jnp.minimum(c * _TGRID + g,
                                           N_VOCAB // _TW)))
        for c in range(_SLABS)
    ],
    out_specs=pl.BlockSpec((_TW, _SLABS * K_DIM), lambda g: (g, 0)),
    out_shape=jax.ShapeDtypeStruct((_SLAB, _SLABS * K_DIM), jnp.float32),
)


@functools.partial(
    pl.kernel,
    out_type=jax.ShapeDtypeStruct((BATCH,), jnp.float32),
    mesh=_mesh,
    compiler_params=pltpu.CompilerParams(
        needs_layout_passes=False, use_tc_tiling_on_sc=False),
    scratch_types=[
        pltpu.VMEM((_IDX_PER_W,), jnp.int32),
        pltpu.VMEM((_IDX_PER_STRIP, 4 * K_DIM), jnp.float32),
        pltpu.VMEM((_IDX_PER_STRIP, 4 * K_DIM), jnp.float32),
        pltpu.VMEM((_IDX_PER_STRIP,), jnp.int32),
        pltpu.VMEM((_IDX_PER_STRIP,), jnp.int32),
        pltpu.VMEM((_IDX_PER_W,), jnp.float32),
        pltpu.VMEM((_ROWS_PER_W,), jnp.float32),
        pltpu.VMEM((16,), jnp.float32),
        pltpu.SemaphoreType.DMA,
        pltpu.SemaphoreType.DMA,
        pltpu.SemaphoreType.DMA,
    ],
)
def _fm_sc(idx_hbm, w1t_hbm, vp_hbm, w0_hbm, out_hbm,
           idx_v, buf_a, buf_b, qid_a, qid_b, w1_v, out_v, w0_v,
           sem_a, sem_b, sem_w):
    wid = lax.axis_index("s") * _NC + lax.axis_index("c")
    base = wid * _IDX_PER_W

    pltpu.sync_copy(w0_hbm, w0_v)
    pltpu.sync_copy(idx_hbm.at[pl.ds(base, _IDX_PER_W)], idx_v)
    cp_w = pltpu.async_copy(w1t_hbm.at[0].at[idx_v], w1_v, sem_w)

    bufs = (buf_a, buf_b)
    qids = (qid_a, qid_b)
    sems = (sem_a, sem_b)

    def stage(c):
        qid = qids[c % 2]

        def qbody(i, _):
            o = i * 16
            qid[pl.ds(o, 16)] = jnp.bitwise_and(
                idx_v[pl.ds(c * _IDX_PER_STRIP + o, 16)], _SLAB - 1)
            return 0

        lax.fori_loop(0, _IDX_PER_STRIP // 16, qbody, 0)
        return pltpu.async_copy(vp_hbm.at[qid], bufs[c % 2], sems[c % 2])

    cp = {0: stage(0)}
    cp_w.wait()

    w0 = w0_v[...]
    lane = lax.broadcasted_iota(jnp.int32, (16,), 0)
    lane26 = lane * N_FIELDS
    zero16 = jnp.zeros((16,), jnp.float32)

    for c in range(_STRIPS):
        cp[c].wait()
        if c + 1 < _STRIPS:
            cp[c + 1] = stage(c + 1)
        buf = bufs[c % 2]

        acc = zero16
        lv = zero16
        for h in range(2):
            def f_body(f, carry):
                s = list(carry[0])
                q = carry[1]
                l = carry[2]
                idx0 = lane26 + f
                raw = plsc.load_gather(idx_v, [c * _IDX_PER_STRIP + idx0])
                off = lax.shift_left(
                    lax.shift_right_logical(raw, 18), 5) + h * 16
                for k in range(16):
                    val = plsc.load_gather(buf, [idx0, off + k])
                    q = q + val * val
                    s[k] = s[k] + val
                if h == 0:
                    l = l + plsc.load_gather(
                        w1_v, [c * _IDX_PER_STRIP + idx0])
                return (tuple(s), q, l)

            s, q, lv = lax.fori_loop(
                0, N_FIELDS, f_body, ((zero16,) * 16, zero16, lv))
            acc = acc - q
            for k in range(16):
                acc = acc + s[k] * s[k]

        out_v[pl.ds(c * 16, 16)] = lv + w0 + 0.5 * acc

    pltpu.sync_copy(out_v, out_hbm.at[pl.ds(wid * _ROWS_PER_W, _ROWS_PER_W)])


def kernel(inputs, W1, V, W0):
    idx = inputs.reshape(-1).astype(jnp.int32)
    w0b = jnp.broadcast_to(W0, (16,))
    vt = jnp.swapaxes(V, 0, 1)
    vp = _tc_prep(vt, vt, vt, vt)
    out = _fm_sc(idx, jnp.swapaxes(W1, 0, 1), vp, w0b)
    return out.reshape(BATCH, 1)


# speedup vs baseline: 13.3977x; 1.0715x over previous
import functools

import jax
import jax.numpy as jnp
from jax import lax
from jax.experimental import pallas as pl
from jax.experimental.pallas import tpu as pltpu
from jax.experimental.pallas import tpu_sc as plsc

N_VOCAB = 1000000
K_DIM = 32
BATCH = 4096
N_FIELDS = 26

_SLABS = 4
_SLAB = 1 << 18
_NC = 2
_NS = 16
_NW = _NC * _NS
_ROWS_PER_W = BATCH // _NW
_IDX_PER_W = _ROWS_PER_W * N_FIELDS
_STRIPS = _ROWS_PER_W // 16
_IDX_PER_STRIP = 16 * N_FIELDS

_mesh = plsc.VectorSubcoreMesh(core_axis_name="c", subcore_axis_name="s")

_TW = 8192
_TGRID = _SLAB // _TW


def _prep_body(v0_ref, v1_ref, v2_ref, v3_ref, vp_ref):
    stacked = jnp.concatenate(
        [ref[...] for ref in (v0_ref, v1_ref, v2_ref, v3_ref)], axis=0)
    vp_ref[...] = stacked.T


_tc_prep = pl.pallas_call(
    _prep_body,
    grid=(_TGRID,),
    in_specs=[
        pl.BlockSpec(
            (K_DIM, _TW),
            lambda g, c=c: (0, jnp.minimum(c * _TGRID + g,
                                           N_VOCAB // _TW)))
        for c in range(_SLABS)
    ],
    out_specs=pl.BlockSpec((_TW, _SLABS * K_DIM), lambda g: (g, 0)),
    out_shape=jax.ShapeDtypeStruct((_SLAB, _SLABS * K_DIM), jnp.float32),
)


@functools.partial(
    pl.kernel,
    out_type=jax.ShapeDtypeStruct((BATCH,), jnp.float32),
    mesh=_mesh,
    compiler_params=pltpu.CompilerParams(
        needs_layout_passes=False, use_tc_tiling_on_sc=False),
    scratch_types=[
        pltpu.VMEM((_IDX_PER_W,), jnp.int32),
        pltpu.VMEM((_IDX_PER_STRIP, 4 * K_DIM), jnp.float32),
        pltpu.VMEM((_IDX_PER_STRIP, 4 * K_DIM), jnp.float32),
        pltpu.VMEM((_IDX_PER_STRIP,), jnp.int32),
        pltpu.VMEM((_IDX_PER_STRIP,), jnp.int32),
        pltpu.VMEM((_IDX_PER_W,), jnp.float32),
        pltpu.VMEM((_ROWS_PER_W,), jnp.float32),
        pltpu.VMEM((16,), jnp.float32),
        pltpu.SemaphoreType.DMA,
        pltpu.SemaphoreType.DMA,
        pltpu.SemaphoreType.DMA,
    ],
)
def _fm_sc(idx_hbm, w1t_hbm, vp_hbm, w0_hbm, out_hbm,
           idx_v, buf_a, buf_b, qid_a, qid_b, w1_v, out_v, w0_v,
           sem_a, sem_b, sem_w):
    wid = lax.axis_index("s") * _NC + lax.axis_index("c")
    base = wid * _IDX_PER_W

    pltpu.sync_copy(w0_hbm, w0_v)
    pltpu.sync_copy(idx_hbm.at[pl.ds(base, _IDX_PER_W)], idx_v)
    cp_w = pltpu.async_copy(w1t_hbm.at[0].at[idx_v], w1_v, sem_w)

    bufs = (buf_a, buf_b)
    qids = (qid_a, qid_b)
    sems = (sem_a, sem_b)

    def stage(c):
        qid = qids[c % 2]

        def qbody(i, _):
            o = i * 16
            qid[pl.ds(o, 16)] = jnp.bitwise_and(
                idx_v[pl.ds(c * _IDX_PER_STRIP + o, 16)], _SLAB - 1)
            return 0

        lax.fori_loop(0, _IDX_PER_STRIP // 16, qbody, 0)
        return pltpu.async_copy(vp_hbm.at[qid], bufs[c % 2], sems[c % 2])

    cp = {0: stage(0)}
    cp_w.wait()

    w0 = w0_v[...]
    lane = lax.broadcasted_iota(jnp.int32, (16,), 0)
    lane26 = lane * N_FIELDS
    zero16 = jnp.zeros((16,), jnp.float32)

    for c in range(_STRIPS):
        cp[c].wait()
        if c + 1 < _STRIPS:
            cp[c + 1] = stage(c + 1)
        buf = bufs[c % 2]

        acc = zero16
        lv = zero16
        for h in range(2):
            def f_body(f, carry):
                s = list(carry[0])
                q = carry[1]
                l = carry[2]
                idx0 = lane26 + f
                raw = plsc.load_gather(idx_v, [c * _IDX_PER_STRIP + idx0])
                off = lax.shift_left(
                    lax.shift_right_logical(raw, 18), 5) + h * 16
                for k in range(16):
                    val = plsc.load_gather(buf, [idx0, off + k])
                    q = q + val * val
                    s[k] = s[k] + val
                if h == 0:
                    l = l + plsc.load_gather(
                        w1_v, [c * _IDX_PER_STRIP + idx0])
                return (tuple(s), q, l)

            s, q, lv = lax.fori_loop(
                0, N_FIELDS, f_body, ((zero16,) * 16, zero16, lv))
            acc = acc - q
            for k in range(16):
                acc = acc + s[k] * s[k]

        out_v[pl.ds(c * 16, 16)] = lv + w0 + 0.5 * acc

    pltpu.sync_copy(out_v, out_hbm.at[pl.ds(wid * _ROWS_PER_W, _ROWS_PER_W)])


def kernel(inputs, W1, V, W0):
    idx = inputs.reshape(-1).astype(jnp.int32)
    w0b = jnp.broadcast_to(W0, (16,))
    vt = jnp.swapaxes(V, 0, 1)
    vp = _tc_prep(vt, vt, vt, vt)
    out = _fm_sc(idx, jnp.swapaxes(W1, 0, 1), vp, w0b)
    return out.reshape(BATCH, 1)
